# baseline (speedup 1.0000x reference)
import functools

import jax
import jax.numpy as jnp
from jax import lax
from jax.experimental import pallas as pl
from jax.experimental.pallas import tpu as pltpu
from jax.experimental.pallas import tpu_sc as plsc

N = 10000
R = 10112
ROW_BLK = 200
NC = 2
NS = 16
B = 64
D = 128
AGG_FRAC0 = 0.95
NBUF = 4


def _make_deg(e_pad):
  per_tile = e_pad // (NC * NS)
  batches = per_tile // B
  rps = R // NS
  mesh = plsc.VectorSubcoreMesh(core_axis_name="c", subcore_axis_name="s")

  @functools.partial(
      pl.kernel, mesh=mesh,
      out_type=[jax.ShapeDtypeStruct((R, D), jnp.float32),
                jax.ShapeDtypeStruct((R, D), jnp.float32)],
      scratch_types=[
          pltpu.VMEM((B,), jnp.int32),
          pltpu.VMEM((B, D), jnp.float32),
          pltpu.VMEM_SHARED((R, D), jnp.float32),
      ],
  )
  def deg_kernel(dst_hbm, zero_hbm, ones_hbm, out_a, out_b, dst_v, ones_v, acc_sh):
    c = lax.axis_index("c")
    s = lax.axis_index("s")
    wid = s * NC + c
    r0 = s * rps
    pltpu.sync_copy(ones_hbm, ones_v)
    pltpu.sync_copy(zero_hbm.at[pl.ds(r0, rps)], acc_sh.at[pl.ds(r0, rps)])
    plsc.subcore_barrier()
    base = wid * per_tile

    def body(i, carry):
      pltpu.sync_copy(dst_hbm.at[pl.ds(base + i * B, B)], dst_v)
      pltpu.sync_copy(ones_v, acc_sh.at[dst_v], add=True)
      return carry

    lax.fori_loop(0, batches, body, 0)
    plsc.subcore_barrier()

    @pl.when(c == 0)
    def _():
      pltpu.sync_copy(acc_sh.at[pl.ds(r0, rps)], out_a.at[pl.ds(r0, rps)])

    @pl.when(c == 1)
    def _():
      pltpu.sync_copy(acc_sh.at[pl.ds(r0, rps)], out_b.at[pl.ds(r0, rps)])

  return deg_kernel


def _make_agg(e_pad, frac0=0.5):
  total_b = e_pad // (NS * B)
  nb0 = max(NBUF, int(round(total_b * frac0 / NBUF)) * NBUF)
  nb1 = total_b - nb0
  rps = R // NS
  mesh = plsc.VectorSubcoreMesh(core_axis_name="c", subcore_axis_name="s")

  @functools.partial(
      pl.kernel, mesh=mesh,
      out_type=[jax.ShapeDtypeStruct((R, D), jnp.float32),
                jax.ShapeDtypeStruct((R, D), jnp.float32)],
      scratch_types=[
          pltpu.VMEM((NBUF, B), jnp.int32),
          pltpu.VMEM((NBUF, B), jnp.int32),
          pltpu.VMEM((NBUF, B, D), jnp.float32),
          pltpu.VMEM_SHARED((R, D), jnp.float32),
      ] + [pltpu.SemaphoreType.DMA] * NBUF,
  )
  def agg_kernel(g_hbm, src_hbm, dst_hbm, zero_hbm, out_a, out_b,
                 src_v, dst_v, rows_v, acc_sh, *sems):
    c = lax.axis_index("c")
    s = lax.axis_index("s")
    r0 = s * rps
    pltpu.sync_copy(zero_hbm.at[pl.ds(r0, rps)], acc_sh.at[pl.ds(r0, rps)])
    plsc.subcore_barrier()
    nb = jnp.where(c == 0, nb0, nb1)
    base = jnp.where(c == 0, s * nb0, NS * nb0 + s * nb1) * B

    def prefetch(i, slot):
      off = base + i * B
      pltpu.sync_copy(src_hbm.at[pl.ds(off, B)], src_v.at[slot])
      pltpu.sync_copy(dst_hbm.at[pl.ds(off, B)], dst_v.at[slot])
      pltpu.async_copy(g_hbm.at[src_v.at[slot]], rows_v.at[slot], sems[slot])

    for b in range(NBUF - 1):
      @pl.when(b < nb)
      def _(b=b):
        prefetch(b, b)

    def round_body(g, carry):
      for b in range(NBUF):
        i = NBUF * g + b
        nxt = i + (NBUF - 1)

        @pl.when(nxt < nb)
        def _():
          prefetch(nxt, (b + NBUF - 1) % NBUF)

        pltpu.make_async_copy(g_hbm.at[src_v.at[b]], rows_v.at[b],
                              sems[b]).wait()
        pltpu.sync_copy(rows_v.at[b], acc_sh.at[dst_v.at[b]], add=True)
      return carry

    lax.fori_loop(0, nb // NBUF, round_body, 0)
    plsc.subcore_barrier()

    @pl.when(c == 0)
    def _():
      pltpu.sync_copy(acc_sh.at[pl.ds(r0, rps)], out_a.at[pl.ds(r0, rps)])

    @pl.when(c == 1)
    def _():
      pltpu.sync_copy(acc_sh.at[pl.ds(r0, rps)], out_b.at[pl.ds(r0, rps)])

  return agg_kernel


def _tc1_body(da_ref, db_ref, x_ref, w1_ref, g1_ref, dinv_ref):
  deg = da_ref[:, 0:1] + db_ref[:, 0:1] + 1.0
  dinv = lax.rsqrt(deg)
  h = jnp.dot(x_ref[...], w1_ref[...], preferred_element_type=jnp.float32)
  g1_ref[...] = h * dinv
  dinv_ref[...] = jnp.broadcast_to(dinv, (ROW_BLK, 128))


def _tc2_body(aa_ref, ab_ref, g1_ref, dinv_ref, b1_ref, w2_ref, g2_ref):
  dinv = dinv_ref[:, 0:1]
  s = dinv * (aa_ref[...] + ab_ref[...] + g1_ref[...]) + b1_ref[...]
  o = jnp.maximum(s, 0.0)
  h2 = jnp.dot(o, w2_ref[...], preferred_element_type=jnp.float32)
  g2_ref[...] = jnp.concatenate(
      [h2 * dinv, jnp.zeros((ROW_BLK, 64), jnp.float32)], axis=1)


def _tc3_body(aa_ref, ab_ref, g2_ref, dinv_ref, b2_ref, out_ref):
  dinv = dinv_ref[:, 0:1]
  z = dinv * (aa_ref[:, :64] + ab_ref[:, :64] + g2_ref[:, :64]) + b2_ref[...]
  m = jnp.max(z, axis=1, keepdims=True)
  lse = jnp.log(jnp.sum(jnp.exp(z - m), axis=1, keepdims=True)) + m
  out_ref[...] = z - lse


def _row_spec(d, off_blocks=0):
  return pl.BlockSpec((ROW_BLK, d), lambda i, o=off_blocks: (i + o, 0))


def _full_spec(r, c):
  return pl.BlockSpec((r, c), lambda i: (0, 0))


def kernel(x, edge_index, W1, b1, W2, b2):
  e = edge_index.shape[1]
  e_pad = -(-e // (NS * B * NBUF * 2)) * (NS * B * NBUF * 2)
  src = edge_index[0].astype(jnp.int32)
  dst = edge_index[1].astype(jnp.int32)
  pad = e_pad - e
  src_p = jnp.concatenate([src, jnp.zeros((pad,), jnp.int32)])
  dst_p = jnp.concatenate([dst, jnp.full((pad,), N, jnp.int32)])

  z128 = jnp.zeros((R, D), jnp.float32)
  ones128 = jnp.ones((B, D), jnp.float32)

  dga, dgb = _make_deg(e_pad)(dst_p, z128, ones128)

  grid = (N // ROW_BLK,)
  g1, dinv = pl.pallas_call(
      _tc1_body,
      grid=grid,
      in_specs=[_row_spec(D), _row_spec(D),
                _row_spec(128), _full_spec(128, 128)],
      out_specs=[_row_spec(128), _row_spec(128)],
      out_shape=[jax.ShapeDtypeStruct((N, 128), jnp.float32),
                 jax.ShapeDtypeStruct((N, 128), jnp.float32)],
  )(dga, dgb, x, W1)

  a1a, a1b = _make_agg(e_pad, AGG_FRAC0)(g1, src_p, dst_p, z128)

  g2 = pl.pallas_call(
      _tc2_body,
      grid=grid,
      in_specs=[_row_spec(D), _row_spec(D), _row_spec(128),
                _row_spec(128), _full_spec(1, 128), _full_spec(128, 64)],
      out_specs=_row_spec(128),
      out_shape=jax.ShapeDtypeStruct((N, 128), jnp.float32),
  )(a1a, a1b, g1, dinv, b1.reshape(1, 128), W2)

  a2a, a2b = _make_agg(e_pad, AGG_FRAC0)(g2, src_p, dst_p, z128)

  out = pl.pallas_call(
      _tc3_body,
      grid=grid,
      in_specs=[_row_spec(D), _row_spec(D), _row_spec(128),
                _row_spec(128), _full_spec(1, 64)],
      out_specs=_row_spec(64),
      out_shape=jax.ShapeDtypeStruct((N, 64), jnp.float32),
  )(a2a, a2b, g2, dinv, b2.reshape(1, 64))

  return out


# speedup vs baseline: 1.1091x; 1.1091x over previous
import functools

import jax
import jax.numpy as jnp
from jax import lax
from jax.experimental import pallas as pl
from jax.experimental.pallas import tpu as pltpu
from jax.experimental.pallas import tpu_sc as plsc

N = 10000
R = 10112
ROW_BLK = 200
NC = 2
NS = 16
B = 96
D = 128
AGG_FRAC0 = 0.95
NBUF = 2


def _make_deg(e_pad):
  per_tile = e_pad // (NC * NS)
  batches = per_tile // B
  rps = R // NS
  mesh = plsc.VectorSubcoreMesh(core_axis_name="c", subcore_axis_name="s")

  @functools.partial(
      pl.kernel, mesh=mesh,
      out_type=[jax.ShapeDtypeStruct((R, D), jnp.float32),
                jax.ShapeDtypeStruct((R, D), jnp.float32)],
      scratch_types=[
          pltpu.VMEM((B,), jnp.int32),
          pltpu.VMEM((B, D), jnp.float32),
          pltpu.VMEM_SHARED((R, D), jnp.float32),
      ],
  )
  def deg_kernel(dst_hbm, zero_hbm, ones_hbm, out_a, out_b, dst_v, ones_v, acc_sh):
    c = lax.axis_index("c")
    s = lax.axis_index("s")
    wid = s * NC + c
    r0 = s * rps
    pltpu.sync_copy(ones_hbm, ones_v)
    pltpu.sync_copy(zero_hbm.at[pl.ds(r0, rps)], acc_sh.at[pl.ds(r0, rps)])
    plsc.subcore_barrier()
    base = wid * per_tile

    def body(i, carry):
      pltpu.sync_copy(dst_hbm.at[pl.ds(base + i * B, B)], dst_v)
      pltpu.sync_copy(ones_v, acc_sh.at[dst_v], add=True)
      return carry

    lax.fori_loop(0, batches, body, 0)
    plsc.subcore_barrier()

    @pl.when(c == 0)
    def _():
      pltpu.sync_copy(acc_sh.at[pl.ds(r0, rps)], out_a.at[pl.ds(r0, rps)])

    @pl.when(c == 1)
    def _():
      pltpu.sync_copy(acc_sh.at[pl.ds(r0, rps)], out_b.at[pl.ds(r0, rps)])

  return deg_kernel


def _make_agg(e_pad, frac0=0.5):
  total_b = e_pad // (NS * B)
  nb0 = max(NBUF, int(round(total_b * frac0 / NBUF)) * NBUF)
  nb1 = total_b - nb0
  rps = R // NS
  mesh = plsc.VectorSubcoreMesh(core_axis_name="c", subcore_axis_name="s")

  @functools.partial(
      pl.kernel, mesh=mesh,
      out_type=[jax.ShapeDtypeStruct((R, D), jnp.float32),
                jax.ShapeDtypeStruct((R, D), jnp.float32)],
      scratch_types=[
          pltpu.VMEM((NBUF, B), jnp.int32),
          pltpu.VMEM((NBUF, B), jnp.int32),
          pltpu.VMEM((NBUF, B, D), jnp.float32),
          pltpu.VMEM_SHARED((R, D), jnp.float32),
      ] + [pltpu.SemaphoreType.DMA] * NBUF,
  )
  def agg_kernel(g_hbm, src_hbm, dst_hbm, zero_hbm, out_a, out_b,
                 src_v, dst_v, rows_v, acc_sh, *sems):
    c = lax.axis_index("c")
    s = lax.axis_index("s")
    r0 = s * rps
    pltpu.sync_copy(zero_hbm.at[pl.ds(r0, rps)], acc_sh.at[pl.ds(r0, rps)])
    plsc.subcore_barrier()
    nb = jnp.where(c == 0, nb0, nb1)
    base = jnp.where(c == 0, s * nb0, NS * nb0 + s * nb1) * B

    def prefetch(i, slot):
      off = base + i * B
      pltpu.sync_copy(src_hbm.at[pl.ds(off, B)], src_v.at[slot])
      pltpu.sync_copy(dst_hbm.at[pl.ds(off, B)], dst_v.at[slot])
      pltpu.async_copy(g_hbm.at[src_v.at[slot]], rows_v.at[slot], sems[slot])

    for b in range(NBUF - 1):
      @pl.when(b < nb)
      def _(b=b):
        prefetch(b, b)

    def round_body(g, carry):
      for b in range(NBUF):
        i = NBUF * g + b
        nxt = i + (NBUF - 1)

        @pl.when(nxt < nb)
        def _():
          prefetch(nxt, (b + NBUF - 1) % NBUF)

        pltpu.make_async_copy(g_hbm.at[src_v.at[b]], rows_v.at[b],
                              sems[b]).wait()
        pltpu.sync_copy(rows_v.at[b], acc_sh.at[dst_v.at[b]], add=True)
      return carry

    lax.fori_loop(0, nb // NBUF, round_body, 0)
    plsc.subcore_barrier()

    @pl.when(c == 0)
    def _():
      pltpu.sync_copy(acc_sh.at[pl.ds(r0, rps)], out_a.at[pl.ds(r0, rps)])

    @pl.when(c == 1)
    def _():
      pltpu.sync_copy(acc_sh.at[pl.ds(r0, rps)], out_b.at[pl.ds(r0, rps)])

  return agg_kernel


def _tc1_body(da_ref, db_ref, x_ref, w1_ref, g1_ref, dinv_ref):
  deg = da_ref[:, 0:1] + db_ref[:, 0:1] + 1.0
  dinv = lax.rsqrt(deg)
  h = jnp.dot(x_ref[...], w1_ref[...], preferred_element_type=jnp.float32)
  g1_ref[...] = h * dinv
  dinv_ref[...] = jnp.broadcast_to(dinv, (ROW_BLK, 128))


def _tc2_body(aa_ref, ab_ref, g1_ref, dinv_ref, b1_ref, w2_ref, g2_ref):
  dinv = dinv_ref[:, 0:1]
  s = dinv * (aa_ref[...] + ab_ref[...] + g1_ref[...]) + b1_ref[...]
  o = jnp.maximum(s, 0.0)
  h2 = jnp.dot(o, w2_ref[...], preferred_element_type=jnp.float32)
  g2_ref[...] = jnp.concatenate(
      [h2 * dinv, jnp.zeros((ROW_BLK, 64), jnp.float32)], axis=1)


def _tc3_body(aa_ref, ab_ref, g2_ref, dinv_ref, b2_ref, out_ref):
  dinv = dinv_ref[:, 0:1]
  z = dinv * (aa_ref[:, :64] + ab_ref[:, :64] + g2_ref[:, :64]) + b2_ref[...]
  m = jnp.max(z, axis=1, keepdims=True)
  lse = jnp.log(jnp.sum(jnp.exp(z - m), axis=1, keepdims=True)) + m
  out_ref[...] = z - lse


def _row_spec(d, off_blocks=0):
  return pl.BlockSpec((ROW_BLK, d), lambda i, o=off_blocks: (i + o, 0))


def _full_spec(r, c):
  return pl.BlockSpec((r, c), lambda i: (0, 0))


def kernel(x, edge_index, W1, b1, W2, b2):
  e = edge_index.shape[1]
  e_pad = -(-e // (NS * B * NBUF * 2)) * (NS * B * NBUF * 2)
  src = edge_index[0].astype(jnp.int32)
  dst = edge_index[1].astype(jnp.int32)
  pad = e_pad - e
  src_p = jnp.concatenate([src, jnp.zeros((pad,), jnp.int32)])
  dst_p = jnp.concatenate([dst, jnp.full((pad,), N, jnp.int32)])

  z128 = jnp.zeros((R, D), jnp.float32)
  ones128 = jnp.ones((B, D), jnp.float32)

  dga, dgb = _make_deg(e_pad)(dst_p, z128, ones128)

  grid = (N // ROW_BLK,)
  g1, dinv = pl.pallas_call(
      _tc1_body,
      grid=grid,
      in_specs=[_row_spec(D), _row_spec(D),
                _row_spec(128), _full_spec(128, 128)],
      out_specs=[_row_spec(128), _row_spec(128)],
      out_shape=[jax.ShapeDtypeStruct((N, 128), jnp.float32),
                 jax.ShapeDtypeStruct((N, 128), jnp.float32)],
  )(dga, dgb, x, W1)

  a1a, a1b = _make_agg(e_pad, AGG_FRAC0)(g1, src_p, dst_p, z128)

  g2 = pl.pallas_call(
      _tc2_body,
      grid=grid,
      in_specs=[_row_spec(D), _row_spec(D), _row_spec(128),
                _row_spec(128), _full_spec(1, 128), _full_spec(128, 64)],
      out_specs=_row_spec(128),
      out_shape=jax.ShapeDtypeStruct((N, 128), jnp.float32),
  )(a1a, a1b, g1, dinv, b1.reshape(1, 128), W2)

  a2a, a2b = _make_agg(e_pad, AGG_FRAC0)(g2, src_p, dst_p, z128)

  out = pl.pallas_call(
      _tc3_body,
      grid=grid,
      in_specs=[_row_spec(D), _row_spec(D), _row_spec(128),
                _row_spec(128), _full_spec(1, 64)],
      out_specs=_row_spec(64),
      out_shape=jax.ShapeDtypeStruct((N, 64), jnp.float32),
  )(a2a, a2b, g2, dinv, b2.reshape(1, 64))

  return out


# speedup vs baseline: 1.2811x; 1.1551x over previous
import functools

import jax
import jax.numpy as jnp
from jax import lax
from jax.experimental import pallas as pl
from jax.experimental.pallas import tpu as pltpu
from jax.experimental.pallas import tpu_sc as plsc

N = 10000
R = 10112
ROW_BLK = 200
NC = 2
NS = 16
B = 96
D = 128
AGG_FRAC0 = 0.95
NBUF = 3


def _make_deg(e_pad):
  per_tile = e_pad // (NC * NS)
  batches = per_tile // B
  rps = R // NS
  mesh = plsc.VectorSubcoreMesh(core_axis_name="c", subcore_axis_name="s")

  @functools.partial(
      pl.kernel, mesh=mesh,
      out_type=[jax.ShapeDtypeStruct((R, D), jnp.float32),
                jax.ShapeDtypeStruct((R, D), jnp.float32)],
      scratch_types=[
          pltpu.VMEM((B,), jnp.int32),
          pltpu.VMEM((B, D), jnp.float32),
          pltpu.VMEM_SHARED((R, D), jnp.float32),
      ],
  )
  def deg_kernel(dst_hbm, zero_hbm, ones_hbm, out_a, out_b, dst_v, ones_v, acc_sh):
    c = lax.axis_index("c")
    s = lax.axis_index("s")
    wid = s * NC + c
    r0 = s * rps
    pltpu.sync_copy(ones_hbm, ones_v)
    pltpu.sync_copy(zero_hbm.at[pl.ds(r0, rps)], acc_sh.at[pl.ds(r0, rps)])
    plsc.subcore_barrier()
    base = wid * per_tile

    def body(i, carry):
      pltpu.sync_copy(dst_hbm.at[pl.ds(base + i * B, B)], dst_v)
      pltpu.sync_copy(ones_v, acc_sh.at[dst_v], add=True)
      return carry

    lax.fori_loop(0, batches, body, 0)
    plsc.subcore_barrier()

    @pl.when(c == 0)
    def _():
      pltpu.sync_copy(acc_sh.at[pl.ds(r0, rps)], out_a.at[pl.ds(r0, rps)])

    @pl.when(c == 1)
    def _():
      pltpu.sync_copy(acc_sh.at[pl.ds(r0, rps)], out_b.at[pl.ds(r0, rps)])

  return deg_kernel


def _make_agg(e_pad, frac0=0.5):
  total_b = e_pad // (NS * B)
  nb0 = max(NBUF, int(round(total_b * frac0 / NBUF)) * NBUF)
  nb1 = total_b - nb0
  rps = R // NS
  mesh = plsc.VectorSubcoreMesh(core_axis_name="c", subcore_axis_name="s")

  @functools.partial(
      pl.kernel, mesh=mesh,
      out_type=[jax.ShapeDtypeStruct((R, D), jnp.float32),
                jax.ShapeDtypeStruct((R, D), jnp.float32)],
      scratch_types=[
          pltpu.VMEM((NBUF, B), jnp.int32),
          pltpu.VMEM((NBUF, B), jnp.int32),
          pltpu.VMEM((NBUF, B, D), jnp.float32),
          pltpu.VMEM_SHARED((R, D), jnp.float32),
      ] + [pltpu.SemaphoreType.DMA] * NBUF,
  )
  def agg_kernel(g_hbm, src_hbm, dst_hbm, zero_hbm, out_a, out_b,
                 src_v, dst_v, rows_v, acc_sh, *sems):
    c = lax.axis_index("c")
    s = lax.axis_index("s")
    r0 = s * rps
    pltpu.sync_copy(zero_hbm.at[pl.ds(r0, rps)], acc_sh.at[pl.ds(r0, rps)])
    plsc.subcore_barrier()
    nb = jnp.where(c == 0, nb0, nb1)
    base = jnp.where(c == 0, s * nb0, NS * nb0 + s * nb1) * B

    def prefetch(i, slot):
      off = base + i * B
      pltpu.sync_copy(src_hbm.at[pl.ds(off, B)], src_v.at[slot])
      pltpu.sync_copy(dst_hbm.at[pl.ds(off, B)], dst_v.at[slot])
      pltpu.async_copy(g_hbm.at[src_v.at[slot]], rows_v.at[slot], sems[slot])

    for b in range(NBUF - 1):
      @pl.when(b < nb)
      def _(b=b):
        prefetch(b, b)

    def round_body(g, carry):
      for b in range(NBUF):
        i = NBUF * g + b
        nxt = i + (NBUF - 1)

        @pl.when(nxt < nb)
        def _():
          prefetch(nxt, (b + NBUF - 1) % NBUF)

        pltpu.make_async_copy(g_hbm.at[src_v.at[b]], rows_v.at[b],
                              sems[b]).wait()
        pltpu.sync_copy(rows_v.at[b], acc_sh.at[dst_v.at[b]], add=True)
      return carry

    lax.fori_loop(0, nb // NBUF, round_body, 0)
    plsc.subcore_barrier()

    @pl.when(c == 0)
    def _():
      pltpu.sync_copy(acc_sh.at[pl.ds(r0, rps)], out_a.at[pl.ds(r0, rps)])

    @pl.when(c == 1)
    def _():
      pltpu.sync_copy(acc_sh.at[pl.ds(r0, rps)], out_b.at[pl.ds(r0, rps)])

  return agg_kernel


def _tc1_body(da_ref, db_ref, x_ref, w1_ref, g1_ref, dinv_ref):
  deg = da_ref[:, 0:1] + db_ref[:, 0:1] + 1.0
  dinv = lax.rsqrt(deg)
  h = jnp.dot(x_ref[...], w1_ref[...], preferred_element_type=jnp.float32)
  g1_ref[...] = h * dinv
  dinv_ref[...] = jnp.broadcast_to(dinv, (ROW_BLK, 128))


def _tc2_body(aa_ref, ab_ref, g1_ref, dinv_ref, b1_ref, w2_ref, g2_ref):
  dinv = dinv_ref[:, 0:1]
  s = dinv * (aa_ref[...] + ab_ref[...] + g1_ref[...]) + b1_ref[...]
  o = jnp.maximum(s, 0.0)
  h2 = jnp.dot(o, w2_ref[...], preferred_element_type=jnp.float32)
  g2_ref[...] = jnp.concatenate(
      [h2 * dinv, jnp.zeros((ROW_BLK, 64), jnp.float32)], axis=1)


def _tc3_body(aa_ref, ab_ref, g2_ref, dinv_ref, b2_ref, out_ref):
  dinv = dinv_ref[:, 0:1]
  z = dinv * (aa_ref[:, :64] + ab_ref[:, :64] + g2_ref[:, :64]) + b2_ref[...]
  m = jnp.max(z, axis=1, keepdims=True)
  lse = jnp.log(jnp.sum(jnp.exp(z - m), axis=1, keepdims=True)) + m
  out_ref[...] = z - lse


def _row_spec(d, off_blocks=0):
  return pl.BlockSpec((ROW_BLK, d), lambda i, o=off_blocks: (i + o, 0))


def _full_spec(r, c):
  return pl.BlockSpec((r, c), lambda i: (0, 0))


def kernel(x, edge_index, W1, b1, W2, b2):
  e = edge_index.shape[1]
  e_pad = -(-e // (NS * B * NBUF * 2)) * (NS * B * NBUF * 2)
  src = edge_index[0].astype(jnp.int32)
  dst = edge_index[1].astype(jnp.int32)
  pad = e_pad - e
  src_p = jnp.concatenate([src, jnp.zeros((pad,), jnp.int32)])
  dst_p = jnp.concatenate([dst, jnp.full((pad,), N, jnp.int32)])

  z128 = jnp.zeros((R, D), jnp.float32)
  ones128 = jnp.ones((B, D), jnp.float32)

  dga, dgb = _make_deg(e_pad)(dst_p, z128, ones128)

  grid = (N // ROW_BLK,)
  g1, dinv = pl.pallas_call(
      _tc1_body,
      grid=grid,
      in_specs=[_row_spec(D), _row_spec(D),
                _row_spec(128), _full_spec(128, 128)],
      out_specs=[_row_spec(128), _row_spec(128)],
      out_shape=[jax.ShapeDtypeStruct((N, 128), jnp.float32),
                 jax.ShapeDtypeStruct((N, 128), jnp.float32)],
  )(dga, dgb, x, W1)

  a1a, a1b = _make_agg(e_pad, AGG_FRAC0)(g1, src_p, dst_p, z128)

  g2 = pl.pallas_call(
      _tc2_body,
      grid=grid,
      in_specs=[_row_spec(D), _row_spec(D), _row_spec(128),
                _row_spec(128), _full_spec(1, 128), _full_spec(128, 64)],
      out_specs=_row_spec(128),
      out_shape=jax.ShapeDtypeStruct((N, 128), jnp.float32),
  )(a1a, a1b, g1, dinv, b1.reshape(1, 128), W2)

  a2a, a2b = _make_agg(e_pad, AGG_FRAC0)(g2, src_p, dst_p, z128)

  out = pl.pallas_call(
      _tc3_body,
      grid=grid,
      in_specs=[_row_spec(D), _row_spec(D), _row_spec(128),
                _row_spec(128), _full_spec(1, 64)],
      out_specs=_row_spec(64),
      out_shape=jax.ShapeDtypeStruct((N, 64), jnp.float32),
  )(a2a, a2b, g2, dinv, b2.reshape(1, 64))

  return out
